# baseline (speedup 1.0000x reference)
import functools

import jax
import jax.numpy as jnp
from jax import lax
from jax.experimental import pallas as pl
from jax.experimental.pallas import tpu as pltpu
from jax.experimental.pallas import tpu_sc as plsc

N_NODES = 50000

NC, NS, L = 2, 16, 16
WIN = 128
NROWS = 50176
PAD_ROWS = 128
ACC_ROWS = NROWS + PAD_ROWS
WA, WB = 64, 16


def _payload_mlp(x, typ, common, W1, b1, W2, b2, W3, b3, emb, blk):
    n, d_in = x.shape
    has_common = common is not None
    w3e = jnp.zeros((64, 80), jnp.float32).at[:, 5:69].set(W3)
    b3e = jnp.zeros((1, 80), jnp.float32).at[0, 5:69].set(b3)
    embe = jnp.zeros((5, 80), jnp.float32).at[:, 0:5].set(emb)
    cme = (jnp.zeros((10, 80), jnp.float32)
           .at[:, 69:79].set(jnp.eye(10, dtype=jnp.float32)))

    def body(*refs):
        if has_common:
            x_r, t_r, c_r, w1, bb1, w2, bb2, w3x, bb3, emx, cmx, o_r = refs
        else:
            x_r, t_r, w1, bb1, w2, bb2, w3x, bb3, emx, o_r = refs
        xv = x_r[...]
        h = jnp.maximum(
            jnp.dot(xv, w1[...], preferred_element_type=jnp.float32) + bb1[...], 0.0)
        h = jnp.maximum(
            jnp.dot(h, w2[...], preferred_element_type=jnp.float32) + bb2[...], 0.0)
        pay = jnp.dot(h, w3x[...], preferred_element_type=jnp.float32) + bb3[...]
        tcol = t_r[...].astype(jnp.int32)
        oh = (tcol == lax.broadcasted_iota(jnp.int32, (blk, 5), 1)
              ).astype(jnp.float32)
        pay = pay + jnp.dot(oh, emx[...], preferred_element_type=jnp.float32)
        if has_common:
            pay = pay + jnp.dot(c_r[...], cmx[...],
                                preferred_element_type=jnp.float32)
        o_r[...] = pay

    args = [x, typ.astype(jnp.int32).reshape(n, 1)]
    in_specs = [pl.BlockSpec((blk, d_in), lambda i: (i, 0)),
                pl.BlockSpec((blk, 1), lambda i: (i, 0))]
    if has_common:
        args.append(common)
        in_specs.append(pl.BlockSpec((blk, common.shape[1]), lambda i: (i, 0)))
    consts = [W1, b1.reshape(1, -1), W2, b2.reshape(1, -1), w3e, b3e, embe]
    if has_common:
        consts.append(cme)
    for wmat in consts:
        args.append(wmat)
        in_specs.append(pl.BlockSpec(wmat.shape, lambda i: (0, 0)))
    pay = pl.pallas_call(
        body,
        grid=(n // blk,),
        in_specs=in_specs,
        out_specs=pl.BlockSpec((blk, WA + WB), lambda i: (i, 0)),
        out_shape=jax.ShapeDtypeStruct((n, WA + WB), jnp.float32),
    )(*args)
    return pay


def _prep_edges(src, dst, nseg, segw):
    assert segw % 2 == 0
    e = src.shape[0]
    ep = NC * NS * nseg * segw * WIN
    pad = ep - e
    assert pad >= 0
    src = jnp.concatenate([src.astype(jnp.int32), jnp.zeros((pad,), jnp.int32)])
    dst = jnp.concatenate([dst.astype(jnp.int32),
                           NROWS + (jnp.arange(pad, dtype=jnp.int32) % PAD_ROWS)])
    return (src.reshape(NC, NS, nseg, segw, WIN),
            dst.reshape(NC, NS, nseg, segw, WIN))


def _edge_pass(pay_t, pay_l, pay_c, ts, td, ls, ld, cs, cd, width):
    seg_max = max(ts.shape[3], ls.shape[3], cs.shape[3])
    mesh = plsc.VectorSubcoreMesh(core_axis_name="c", subcore_axis_name="s")

    @functools.partial(
        pl.kernel,
        out_type=jax.ShapeDtypeStruct((NC, ACC_ROWS, width), jnp.float32),
        mesh=mesh,
        compiler_params=pltpu.CompilerParams(use_tc_tiling_on_sc=False),
        scratch_types=[
            pltpu.VMEM_SHARED((ACC_ROWS, width), jnp.float32),
            pltpu.VMEM((seg_max, WIN), jnp.int32),
            pltpu.VMEM((seg_max, WIN), jnp.int32),
            pltpu.VMEM((WIN, width), jnp.float32),
            pltpu.VMEM((WIN, width), jnp.float32),
            pltpu.SemaphoreType.DMA,
            pltpu.SemaphoreType.DMA,
        ],
    )
    def edge_kernel(pt, plh, pc, tsr, tdr, lsr, ldr, csr, cdr, out,
                    acc, srcb, dstb, rows0, rows1, sg0, sg1):
        cid = lax.axis_index("c")
        sid = lax.axis_index("s")
        zero16 = jnp.zeros((L,), jnp.float32)

        @pl.loop(0, WIN)
        def _(r):
            for c in range(width // L):
                rows0[r, pl.ds(c * L, L)] = zero16

        stripe = sid * (ACC_ROWS // NS)
        nfull = (ACC_ROWS // NS) // WIN
        tail = (ACC_ROWS // NS) % WIN

        @pl.loop(0, nfull)
        def _(k):
            pltpu.sync_copy(rows0, acc.at[pl.ds(stripe + k * WIN, WIN)])

        pltpu.sync_copy(rows0.at[pl.ds(0, tail)],
                        acc.at[pl.ds(stripe + nfull * WIN, tail)])

        plsc.subcore_barrier()

        def run_type(pay, s_hbm, d_hbm):
            nseg, segw = s_hbm.shape[2], s_hbm.shape[3]

            def g_start(wi, buf, sem):
                pltpu.make_async_copy(pay.at[srcb.at[wi]], buf, sem).start()

            def g_wait(wi, buf, sem):
                pltpu.make_async_copy(pay.at[srcb.at[wi]], buf, sem).wait()

            def scat(wi, buf):
                pltpu.sync_copy(buf, acc.at[dstb.at[wi]], add=True)

            @pl.loop(0, nseg)
            def _(g):
                pltpu.sync_copy(s_hbm.at[cid, sid, g], srcb.at[pl.ds(0, segw)])
                pltpu.sync_copy(d_hbm.at[cid, sid, g], dstb.at[pl.ds(0, segw)])
                g_start(0, rows0, sg0)
                g_start(1, rows1, sg1)
                m = segw // 2

                @pl.loop(0, m)
                def _(k):
                    w0 = 2 * k
                    w1 = w0 + 1
                    g_wait(w0, rows0, sg0)
                    scat(w0, rows0)

                    @pl.when(k < m - 1)
                    def _():
                        g_start(w0 + 2, rows0, sg0)

                    g_wait(w1, rows1, sg1)
                    scat(w1, rows1)

                    @pl.when(k < m - 1)
                    def _():
                        g_start(w1 + 2, rows1, sg1)

        run_type(pt, tsr, tdr)
        run_type(plh, lsr, ldr)
        run_type(pc, csr, cdr)

        plsc.subcore_barrier()

        opc = ACC_ROWS // NS
        pltpu.sync_copy(acc.at[pl.ds(sid * opc, opc)],
                        out.at[cid, pl.ds(sid * opc, opc)])

    return edge_kernel(pay_t, pay_l, pay_c, ts, td, ls, ld, cs, cd)


def kernel(track_vars, lep_vars, cell_vars, common_tracks, common_leps,
           track_type, lep_type, cell_type,
           t2n_src, t2n_dst, l2n_src, l2n_dst, c2n_src, c2n_dst,
           emb_table,
           tW1, tb1, tW2, tb2, tW3, tb3,
           lW1, lb1, lW2, lb2, lW3, lb3,
           cW1, cb1, cW2, cb2, cW3, cb3):
    pay_t = _payload_mlp(track_vars, track_type, common_tracks,
                         tW1, tb1, tW2, tb2, tW3, tb3, emb_table, 4000)
    pay_l = _payload_mlp(lep_vars, lep_type, common_leps,
                         lW1, lb1, lW2, lb2, lW3, lb3, emb_table, 10000)
    pay_c = _payload_mlp(cell_vars, cell_type, None,
                         cW1, cb1, cW2, cb2, cW3, cb3, emb_table, 4000)
    ts, td = _prep_edges(t2n_src, t2n_dst, 5, 10)
    ls, ld = _prep_edges(l2n_src, l2n_dst, 1, 6)
    cs, cd = _prep_edges(c2n_src, c2n_dst, 14, 14)
    parts = []
    for lo, w in ((0, 32), (32, 32), (64, 16)):
        o = _edge_pass(pay_t[:, lo:lo + w], pay_l[:, lo:lo + w],
                       pay_c[:, lo:lo + w], ts, td, ls, ld, cs, cd, w)
        parts.append(o[0, :N_NODES] + o[1, :N_NODES])
    return jnp.concatenate([parts[0], parts[1], parts[2][:, :15]], axis=1)


# speedup vs baseline: 1.1653x; 1.1653x over previous
import functools

import jax
import jax.numpy as jnp
from jax import lax
from jax.experimental import pallas as pl
from jax.experimental.pallas import tpu as pltpu
from jax.experimental.pallas import tpu_sc as plsc

N_NODES = 50000

NC, NS, L = 2, 16, 16
WIN = 128
ROWS_PER_CORE = 25088
OUT_ROWS = NC * ROWS_PER_CORE
PAD_ROWS = 128
ACC_ROWS = ROWS_PER_CORE + PAD_ROWS
WA, WB = 64, 16
EDGE_ALIGN = NS * WIN * 2


def _payload_mlp(x, typ, common, W1, b1, W2, b2, W3, b3, emb, blk):
    n, d_in = x.shape
    has_common = common is not None
    w3e = jnp.zeros((64, 80), jnp.float32).at[:, 5:69].set(W3)
    b3e = jnp.zeros((1, 80), jnp.float32).at[0, 5:69].set(b3)
    embe = jnp.zeros((5, 80), jnp.float32).at[:, 0:5].set(emb)
    cme = (jnp.zeros((10, 80), jnp.float32)
           .at[:, 69:79].set(jnp.eye(10, dtype=jnp.float32)))

    def body(*refs):
        if has_common:
            x_r, t_r, c_r, w1, bb1, w2, bb2, w3x, bb3, emx, cmx, oa_r, ob_r = refs
        else:
            x_r, t_r, w1, bb1, w2, bb2, w3x, bb3, emx, oa_r, ob_r = refs
        xv = x_r[...]
        h = jnp.maximum(
            jnp.dot(xv, w1[...], preferred_element_type=jnp.float32) + bb1[...], 0.0)
        h = jnp.maximum(
            jnp.dot(h, w2[...], preferred_element_type=jnp.float32) + bb2[...], 0.0)
        pay = jnp.dot(h, w3x[...], preferred_element_type=jnp.float32) + bb3[...]
        tcol = t_r[...]
        oh = (tcol == lax.broadcasted_iota(jnp.int32, (blk, 5), 1)
              ).astype(jnp.float32)
        pay = pay + jnp.dot(oh, emx[...], preferred_element_type=jnp.float32)
        if has_common:
            pay = pay + jnp.dot(c_r[...], cmx[...],
                                preferred_element_type=jnp.float32)
        oa_r[...] = pay[:, :WA]
        ob_r[...] = pay[:, WA:]

    args = [x, typ.astype(jnp.int32).reshape(n, 1)]
    in_specs = [pl.BlockSpec((blk, d_in), lambda i: (i, 0)),
                pl.BlockSpec((blk, 1), lambda i: (i, 0))]
    if has_common:
        args.append(common)
        in_specs.append(pl.BlockSpec((blk, common.shape[1]), lambda i: (i, 0)))
    consts = [W1, b1.reshape(1, -1), W2, b2.reshape(1, -1), w3e, b3e, embe]
    if has_common:
        consts.append(cme)
    for wmat in consts:
        args.append(wmat)
        in_specs.append(pl.BlockSpec(wmat.shape, lambda i: (0, 0)))
    return pl.pallas_call(
        body,
        grid=(n // blk,),
        in_specs=in_specs,
        out_specs=[pl.BlockSpec((blk, WA), lambda i: (i, 0)),
                   pl.BlockSpec((blk, WB), lambda i: (i, 0))],
        out_shape=[jax.ShapeDtypeStruct((n, WA), jnp.float32),
                   jax.ShapeDtypeStruct((n, WB), jnp.float32)],
    )(*args)


def _prep_edges(src, dst, segw):
    e = src.shape[0]
    ep = -(-e // EDGE_ALIGN) * EDGE_ALIGN
    pad = ep - e
    src = jnp.concatenate([src.astype(jnp.int32), jnp.zeros((pad,), jnp.int32)])
    dst = jnp.concatenate([dst.astype(jnp.int32),
                           jnp.full((pad,), OUT_ROWS, jnp.int32)])
    spread = ROWS_PER_CORE + (dst & (PAD_ROWS - 1))
    locs = []
    for c in range(NC):
        lo = c * ROWS_PER_CORE
        inr = (dst >= lo) & (dst < lo + ROWS_PER_CORE)
        locs.append(jnp.where(inr, dst - lo, spread))
    nwin = ep // NS // WIN
    nseg = nwin // segw
    assert nseg * segw == nwin and segw % 2 == 0
    return (src.reshape(NS, nseg, segw, WIN),
            jnp.stack(locs).reshape(NC, NS, nseg, segw, WIN))


def _edge_pass(pay_t, pay_l, pay_c, ts, td, ls, ld, cs, cd, width):
    seg_max = max(ts.shape[2], ls.shape[2], cs.shape[2])
    mesh = plsc.VectorSubcoreMesh(core_axis_name="c", subcore_axis_name="s")

    @functools.partial(
        pl.kernel,
        out_type=jax.ShapeDtypeStruct((OUT_ROWS, width), jnp.float32),
        mesh=mesh,
        compiler_params=pltpu.CompilerParams(use_tc_tiling_on_sc=False),
        scratch_types=[
            pltpu.VMEM_SHARED((ACC_ROWS, width), jnp.float32),
            pltpu.VMEM((seg_max, WIN), jnp.int32),
            pltpu.VMEM((seg_max, WIN), jnp.int32),
            pltpu.VMEM((WIN, width), jnp.float32),
            pltpu.VMEM((WIN, width), jnp.float32),
            pltpu.SemaphoreType.DMA,
            pltpu.SemaphoreType.DMA,
        ],
    )
    def edge_kernel(pt, plh, pc, tsr, tdr, lsr, ldr, csr, cdr, out,
                    acc, srcb, dstb, rows0, rows1, sg0, sg1):
        cid = lax.axis_index("c")
        sid = lax.axis_index("s")
        core_base = cid * ROWS_PER_CORE
        zero16 = jnp.zeros((L,), jnp.float32)

        @pl.loop(0, WIN)
        def _(r):
            for c in range(width // L):
                rows0[r, pl.ds(c * L, L)] = zero16

        stripe = sid * (ACC_ROWS // NS)
        nfull = (ACC_ROWS // NS) // WIN
        tail = (ACC_ROWS // NS) % WIN

        @pl.loop(0, nfull)
        def _(k):
            pltpu.sync_copy(rows0, acc.at[pl.ds(stripe + k * WIN, WIN)])

        pltpu.sync_copy(rows0.at[pl.ds(0, tail)],
                        acc.at[pl.ds(stripe + nfull * WIN, tail)])

        plsc.subcore_barrier()

        def run_type(pay, s_hbm, d_hbm):
            nseg, segw = s_hbm.shape[1], s_hbm.shape[2]

            def g_start(wi, buf, sem):
                pltpu.make_async_copy(pay.at[srcb.at[wi]], buf, sem).start()

            def g_wait(wi, buf, sem):
                pltpu.make_async_copy(pay.at[srcb.at[wi]], buf, sem).wait()

            def scat(wi, buf):
                pltpu.sync_copy(buf, acc.at[dstb.at[wi]], add=True)

            @pl.loop(0, nseg)
            def _(g):
                pltpu.sync_copy(s_hbm.at[sid, g], srcb.at[pl.ds(0, segw)])
                pltpu.sync_copy(d_hbm.at[cid, sid, g], dstb.at[pl.ds(0, segw)])
                g_start(0, rows0, sg0)
                g_start(1, rows1, sg1)
                m = segw // 2

                @pl.loop(0, m)
                def _(k):
                    w0 = 2 * k
                    w1 = w0 + 1
                    g_wait(w0, rows0, sg0)
                    scat(w0, rows0)

                    @pl.when(k < m - 1)
                    def _():
                        g_start(w0 + 2, rows0, sg0)

                    g_wait(w1, rows1, sg1)
                    scat(w1, rows1)

                    @pl.when(k < m - 1)
                    def _():
                        g_start(w1 + 2, rows1, sg1)

        run_type(pt, tsr, tdr)
        run_type(plh, lsr, ldr)
        run_type(pc, csr, cdr)

        plsc.subcore_barrier()

        opc = ROWS_PER_CORE // NS
        pltpu.sync_copy(acc.at[pl.ds(sid * opc, opc)],
                        out.at[pl.ds(core_base + sid * opc, opc)])

    return edge_kernel(pay_t, pay_l, pay_c, ts, td, ls, ld, cs, cd)


def kernel(track_vars, lep_vars, cell_vars, common_tracks, common_leps,
           track_type, lep_type, cell_type,
           t2n_src, t2n_dst, l2n_src, l2n_dst, c2n_src, c2n_dst,
           emb_table,
           tW1, tb1, tW2, tb2, tW3, tb3,
           lW1, lb1, lW2, lb2, lW3, lb3,
           cW1, cb1, cW2, cb2, cW3, cb3):
    pa_t, pb_t = _payload_mlp(track_vars, track_type, common_tracks,
                              tW1, tb1, tW2, tb2, tW3, tb3, emb_table, 4000)
    pa_l, pb_l = _payload_mlp(lep_vars, lep_type, common_leps,
                              lW1, lb1, lW2, lb2, lW3, lb3, emb_table, 10000)
    pa_c, pb_c = _payload_mlp(cell_vars, cell_type, None,
                              cW1, cb1, cW2, cb2, cW3, cb3, emb_table, 4000)
    ts, td = _prep_edges(t2n_src, t2n_dst, 14)
    ls, ld = _prep_edges(l2n_src, l2n_dst, 10)
    cs, cd = _prep_edges(c2n_src, c2n_dst, 14)
    out_a = _edge_pass(pa_t, pa_l, pa_c, ts, td, ls, ld, cs, cd, WA)
    out_b = _edge_pass(pb_t, pb_l, pb_c, ts, td, ls, ld, cs, cd, WB)
    return jnp.concatenate([out_a[:N_NODES], out_b[:N_NODES, :15]], axis=1)


# speedup vs baseline: 1.2180x; 1.0452x over previous
import functools

import jax
import jax.numpy as jnp
from jax import lax
from jax.experimental import pallas as pl
from jax.experimental.pallas import tpu as pltpu
from jax.experimental.pallas import tpu_sc as plsc

N_NODES = 50000

NC, NS, L = 2, 16, 16
WIN = 128
ROWS_PER_CORE = 25088
OUT_ROWS = NC * ROWS_PER_CORE
PAD_ROWS = 128
ACC_ROWS = ROWS_PER_CORE + PAD_ROWS
WA, WB = 64, 16
EDGE_ALIGN = NS * WIN * 2


def _payload_mlp(x, typ, common, W1, b1, W2, b2, W3, b3, emb, blk):
    n, d_in = x.shape
    has_common = common is not None
    w3e = jnp.zeros((64, 80), jnp.float32).at[:, 5:69].set(W3)
    b3e = jnp.zeros((1, 80), jnp.float32).at[0, 5:69].set(b3)
    embe = jnp.zeros((5, 80), jnp.float32).at[:, 0:5].set(emb)
    cme = (jnp.zeros((10, 80), jnp.float32)
           .at[:, 69:79].set(jnp.eye(10, dtype=jnp.float32)))

    def body(*refs):
        if has_common:
            x_r, t_r, c_r, w1, bb1, w2, bb2, w3x, bb3, emx, cmx, oa_r, ob_r = refs
        else:
            x_r, t_r, w1, bb1, w2, bb2, w3x, bb3, emx, oa_r, ob_r = refs
        xv = x_r[...]
        h = jnp.maximum(
            jnp.dot(xv, w1[...], preferred_element_type=jnp.float32) + bb1[...], 0.0)
        h = jnp.maximum(
            jnp.dot(h, w2[...], preferred_element_type=jnp.float32) + bb2[...], 0.0)
        pay = jnp.dot(h, w3x[...], preferred_element_type=jnp.float32) + bb3[...]
        tcol = t_r[...]
        oh = (tcol == lax.broadcasted_iota(jnp.int32, (blk, 5), 1)
              ).astype(jnp.float32)
        pay = pay + jnp.dot(oh, emx[...], preferred_element_type=jnp.float32)
        if has_common:
            pay = pay + jnp.dot(c_r[...], cmx[...],
                                preferred_element_type=jnp.float32)
        oa_r[...] = pay[:, :WA]
        ob_r[...] = pay[:, WA:]

    args = [x, typ.astype(jnp.int32).reshape(n, 1)]
    in_specs = [pl.BlockSpec((blk, d_in), lambda i: (i, 0)),
                pl.BlockSpec((blk, 1), lambda i: (i, 0))]
    if has_common:
        args.append(common)
        in_specs.append(pl.BlockSpec((blk, common.shape[1]), lambda i: (i, 0)))
    consts = [W1, b1.reshape(1, -1), W2, b2.reshape(1, -1), w3e, b3e, embe]
    if has_common:
        consts.append(cme)
    for wmat in consts:
        args.append(wmat)
        in_specs.append(pl.BlockSpec(wmat.shape, lambda i: (0, 0)))
    return pl.pallas_call(
        body,
        grid=(n // blk,),
        in_specs=in_specs,
        out_specs=[pl.BlockSpec((blk, WA), lambda i: (i, 0)),
                   pl.BlockSpec((blk, WB), lambda i: (i, 0))],
        out_shape=[jax.ShapeDtypeStruct((n, WA), jnp.float32),
                   jax.ShapeDtypeStruct((n, WB), jnp.float32)],
    )(*args)


def _prep_edges(src, dst, segw):
    e = src.shape[0]
    ep = -(-e // EDGE_ALIGN) * EDGE_ALIGN
    pad = ep - e
    src = jnp.concatenate([src.astype(jnp.int32), jnp.zeros((pad,), jnp.int32)])
    dst = jnp.concatenate([dst.astype(jnp.int32),
                           jnp.full((pad,), OUT_ROWS, jnp.int32)])
    spread = ROWS_PER_CORE + (dst & (PAD_ROWS - 1))
    locs = []
    for c in range(NC):
        lo = c * ROWS_PER_CORE
        inr = (dst >= lo) & (dst < lo + ROWS_PER_CORE)
        locs.append(jnp.where(inr, dst - lo, spread))
    nwin = ep // NS // WIN
    nseg = nwin // segw
    assert nseg * segw == nwin and segw % 2 == 0
    return (src.reshape(NS, nseg, segw, WIN),
            jnp.stack(locs).reshape(NC, NS, nseg, segw, WIN))


def _prep_edges_split(src, dst, nseg, segw):
    assert segw % 2 == 0
    e = src.shape[0]
    ep = NC * NS * nseg * segw * WIN
    pad = ep - e
    assert pad >= 0
    src = jnp.concatenate([src.astype(jnp.int32), jnp.zeros((pad,), jnp.int32)])
    dst = jnp.concatenate([dst.astype(jnp.int32),
                           OUT_ROWS + (jnp.arange(pad, dtype=jnp.int32)
                                       % PAD_ROWS)])
    return (src.reshape(NC, NS, nseg, segw, WIN),
            dst.reshape(NC, NS, nseg, segw, WIN))


def _edge_pass_split(pay_t, pay_l, pay_c, ts, td, ls, ld, cs, cd, width):
    rows2 = OUT_ROWS + PAD_ROWS
    seg_max = max(ts.shape[3], ls.shape[3], cs.shape[3])
    mesh = plsc.VectorSubcoreMesh(core_axis_name="c", subcore_axis_name="s")

    @functools.partial(
        pl.kernel,
        out_type=jax.ShapeDtypeStruct((NC, rows2, width), jnp.float32),
        mesh=mesh,
        compiler_params=pltpu.CompilerParams(use_tc_tiling_on_sc=False),
        scratch_types=[
            pltpu.VMEM_SHARED((rows2, width), jnp.float32),
            pltpu.VMEM((seg_max, WIN), jnp.int32),
            pltpu.VMEM((seg_max, WIN), jnp.int32),
            pltpu.VMEM((WIN, width), jnp.float32),
            pltpu.VMEM((WIN, width), jnp.float32),
            pltpu.SemaphoreType.DMA,
            pltpu.SemaphoreType.DMA,
        ],
    )
    def edge_kernel(pt, plh, pc, tsr, tdr, lsr, ldr, csr, cdr, out,
                    acc, srcb, dstb, rows0, rows1, sg0, sg1):
        cid = lax.axis_index("c")
        sid = lax.axis_index("s")
        zero16 = jnp.zeros((L,), jnp.float32)

        @pl.loop(0, WIN)
        def _(r):
            for c in range(width // L):
                rows0[r, pl.ds(c * L, L)] = zero16

        stripe = sid * (rows2 // NS)
        nfull = (rows2 // NS) // WIN
        tail = (rows2 // NS) % WIN

        @pl.loop(0, nfull)
        def _(k):
            pltpu.sync_copy(rows0, acc.at[pl.ds(stripe + k * WIN, WIN)])

        pltpu.sync_copy(rows0.at[pl.ds(0, tail)],
                        acc.at[pl.ds(stripe + nfull * WIN, tail)])

        plsc.subcore_barrier()

        def run_type(pay, s_hbm, d_hbm):
            nseg, segw = s_hbm.shape[2], s_hbm.shape[3]

            def g_start(wi, buf, sem):
                pltpu.make_async_copy(pay.at[srcb.at[wi]], buf, sem).start()

            def g_wait(wi, buf, sem):
                pltpu.make_async_copy(pay.at[srcb.at[wi]], buf, sem).wait()

            def scat(wi, buf):
                pltpu.sync_copy(buf, acc.at[dstb.at[wi]], add=True)

            @pl.loop(0, nseg)
            def _(g):
                pltpu.sync_copy(s_hbm.at[cid, sid, g], srcb.at[pl.ds(0, segw)])
                pltpu.sync_copy(d_hbm.at[cid, sid, g], dstb.at[pl.ds(0, segw)])
                g_start(0, rows0, sg0)
                g_start(1, rows1, sg1)
                m = segw // 2

                @pl.loop(0, m)
                def _(k):
                    w0 = 2 * k
                    w1 = w0 + 1
                    g_wait(w0, rows0, sg0)
                    scat(w0, rows0)

                    @pl.when(k < m - 1)
                    def _():
                        g_start(w0 + 2, rows0, sg0)

                    g_wait(w1, rows1, sg1)
                    scat(w1, rows1)

                    @pl.when(k < m - 1)
                    def _():
                        g_start(w1 + 2, rows1, sg1)

        run_type(pt, tsr, tdr)
        run_type(plh, lsr, ldr)
        run_type(pc, csr, cdr)

        plsc.subcore_barrier()

        opc = rows2 // NS
        pltpu.sync_copy(acc.at[pl.ds(sid * opc, opc)],
                        out.at[cid, pl.ds(sid * opc, opc)])

    return edge_kernel(pay_t, pay_l, pay_c, ts, td, ls, ld, cs, cd)


def _edge_pass(pay_t, pay_l, pay_c, ts, td, ls, ld, cs, cd, width):
    seg_max = max(ts.shape[2], ls.shape[2], cs.shape[2])
    mesh = plsc.VectorSubcoreMesh(core_axis_name="c", subcore_axis_name="s")

    @functools.partial(
        pl.kernel,
        out_type=jax.ShapeDtypeStruct((OUT_ROWS, width), jnp.float32),
        mesh=mesh,
        compiler_params=pltpu.CompilerParams(use_tc_tiling_on_sc=False),
        scratch_types=[
            pltpu.VMEM_SHARED((ACC_ROWS, width), jnp.float32),
            pltpu.VMEM((seg_max, WIN), jnp.int32),
            pltpu.VMEM((seg_max, WIN), jnp.int32),
            pltpu.VMEM((WIN, width), jnp.float32),
            pltpu.VMEM((WIN, width), jnp.float32),
            pltpu.SemaphoreType.DMA,
            pltpu.SemaphoreType.DMA,
        ],
    )
    def edge_kernel(pt, plh, pc, tsr, tdr, lsr, ldr, csr, cdr, out,
                    acc, srcb, dstb, rows0, rows1, sg0, sg1):
        cid = lax.axis_index("c")
        sid = lax.axis_index("s")
        core_base = cid * ROWS_PER_CORE
        zero16 = jnp.zeros((L,), jnp.float32)

        @pl.loop(0, WIN)
        def _(r):
            for c in range(width // L):
                rows0[r, pl.ds(c * L, L)] = zero16

        stripe = sid * (ACC_ROWS // NS)
        nfull = (ACC_ROWS // NS) // WIN
        tail = (ACC_ROWS // NS) % WIN

        @pl.loop(0, nfull)
        def _(k):
            pltpu.sync_copy(rows0, acc.at[pl.ds(stripe + k * WIN, WIN)])

        pltpu.sync_copy(rows0.at[pl.ds(0, tail)],
                        acc.at[pl.ds(stripe + nfull * WIN, tail)])

        plsc.subcore_barrier()

        def run_type(pay, s_hbm, d_hbm):
            nseg, segw = s_hbm.shape[1], s_hbm.shape[2]

            def g_start(wi, buf, sem):
                pltpu.make_async_copy(pay.at[srcb.at[wi]], buf, sem).start()

            def g_wait(wi, buf, sem):
                pltpu.make_async_copy(pay.at[srcb.at[wi]], buf, sem).wait()

            def scat(wi, buf):
                pltpu.sync_copy(buf, acc.at[dstb.at[wi]], add=True)

            @pl.loop(0, nseg)
            def _(g):
                pltpu.sync_copy(s_hbm.at[sid, g], srcb.at[pl.ds(0, segw)])
                pltpu.sync_copy(d_hbm.at[cid, sid, g], dstb.at[pl.ds(0, segw)])
                g_start(0, rows0, sg0)
                g_start(1, rows1, sg1)
                m = segw // 2

                @pl.loop(0, m)
                def _(k):
                    w0 = 2 * k
                    w1 = w0 + 1
                    g_wait(w0, rows0, sg0)
                    scat(w0, rows0)

                    @pl.when(k < m - 1)
                    def _():
                        g_start(w0 + 2, rows0, sg0)

                    g_wait(w1, rows1, sg1)
                    scat(w1, rows1)

                    @pl.when(k < m - 1)
                    def _():
                        g_start(w1 + 2, rows1, sg1)

        run_type(pt, tsr, tdr)
        run_type(plh, lsr, ldr)
        run_type(pc, csr, cdr)

        plsc.subcore_barrier()

        opc = ROWS_PER_CORE // NS
        pltpu.sync_copy(acc.at[pl.ds(sid * opc, opc)],
                        out.at[pl.ds(core_base + sid * opc, opc)])

    return edge_kernel(pay_t, pay_l, pay_c, ts, td, ls, ld, cs, cd)


def kernel(track_vars, lep_vars, cell_vars, common_tracks, common_leps,
           track_type, lep_type, cell_type,
           t2n_src, t2n_dst, l2n_src, l2n_dst, c2n_src, c2n_dst,
           emb_table,
           tW1, tb1, tW2, tb2, tW3, tb3,
           lW1, lb1, lW2, lb2, lW3, lb3,
           cW1, cb1, cW2, cb2, cW3, cb3):
    pa_t, pb_t = _payload_mlp(track_vars, track_type, common_tracks,
                              tW1, tb1, tW2, tb2, tW3, tb3, emb_table, 4000)
    pa_l, pb_l = _payload_mlp(lep_vars, lep_type, common_leps,
                              lW1, lb1, lW2, lb2, lW3, lb3, emb_table, 10000)
    pa_c, pb_c = _payload_mlp(cell_vars, cell_type, None,
                              cW1, cb1, cW2, cb2, cW3, cb3, emb_table, 4000)
    ts, td = _prep_edges(t2n_src, t2n_dst, 14)
    ls, ld = _prep_edges(l2n_src, l2n_dst, 10)
    cs, cd = _prep_edges(c2n_src, c2n_dst, 14)
    out_a = _edge_pass(pa_t, pa_l, pa_c, ts, td, ls, ld, cs, cd, WA)
    ts2, td2 = _prep_edges_split(t2n_src, t2n_dst, 5, 10)
    ls2, ld2 = _prep_edges_split(l2n_src, l2n_dst, 1, 6)
    cs2, cd2 = _prep_edges_split(c2n_src, c2n_dst, 14, 14)
    out_b = _edge_pass_split(pb_t, pb_l, pb_c,
                             ts2, td2, ls2, ld2, cs2, cd2, WB)
    ob = out_b[0, :N_NODES] + out_b[1, :N_NODES]
    return jnp.concatenate([out_a[:N_NODES], ob[:, :15]], axis=1)
